# baseline (speedup 1.0000x reference)
import functools

import jax
import jax.numpy as jnp
from jax import lax
from jax.experimental import pallas as pl
from jax.experimental.pallas import tpu as pltpu
from jax.experimental.pallas import tpu_sc as plsc

N = 16384
E = 262144
D = 128
DH = 64
G = 256
SIN = 8192
OUT = 10
NC = 2
NS = 16
CPE = 512
K2OPS = E // NS // CPE
K1OPS = E // (NC * NS) // CPE
RING = 2
DW = 16

_mesh = plsc.VectorSubcoreMesh(core_axis_name="c", subcore_axis_name="s")


def _fill(ref, rows, width, value):
    def body(i, carry):
        for k in range(width // 16):
            ref[i, pl.ds(k * 16, 16)] = jnp.full((16,), value, jnp.float32)
        return carry
    lax.fori_loop(0, rows, body, 0)


@functools.partial(
    pl.kernel,
    out_type=jax.ShapeDtypeStruct((NC, N, 2), jnp.bfloat16),
    mesh=_mesh,
    compiler_params=pltpu.CompilerParams(use_tc_tiling_on_sc=False),
    scratch_types=[
        pltpu.VMEM((K1OPS, 1, CPE), jnp.int32),
        pltpu.VMEM((CPE, DW), jnp.bfloat16),
        pltpu.VMEM((1024, DW), jnp.bfloat16),
        pltpu.VMEM_SHARED((N, DW), jnp.bfloat16),
        pltpu.SemaphoreType.DMA,
    ],
)
def _deg_kernel(dst_hbm, out_hbm, idx_v, ones_v, zbuf, deg_sh, sem0):
    c = lax.axis_index("c")
    s = lax.axis_index("s")
    tid = c * NS + s
    pltpu.sync_copy(dst_hbm.at[tid], idx_v)
    def zb_body(i, carry):
        zbuf[pl.ds(2 * i, 2), :] = jnp.zeros((2, DW), jnp.bfloat16)
        return carry
    lax.fori_loop(0, 512, zb_body, 0)
    pltpu.sync_copy(zbuf, deg_sh.at[pl.ds(s * 1024, 1024)])
    def ones_body(i, carry):
        ones_v[pl.ds(2 * i, 2), :] = jnp.ones((2, DW), jnp.bfloat16)
        return carry
    lax.fori_loop(0, CPE // 2, ones_body, 0)
    plsc.subcore_barrier()
    def body(g, carry):
        ds_ = [pltpu.async_copy(ones_v, deg_sh.at[idx_v.at[4 * g + b, 0]],
                                sem0, add=True) for b in range(4)]
        for d in ds_:
            d.wait()
        return carry
    lax.fori_loop(0, K1OPS // 4, body, 0)
    plsc.subcore_barrier()
    pltpu.sync_copy(deg_sh.at[pl.ds(s * 1024, 1024)], zbuf)
    pltpu.sync_copy(zbuf.at[:, pl.ds(0, 2)], out_hbm.at[c, pl.ds(s * 1024, 1024)])


@functools.partial(
    pl.kernel,
    out_type=jax.ShapeDtypeStruct((NC, N, DH), jnp.bfloat16),
    mesh=_mesh,
    compiler_params=pltpu.CompilerParams(use_tc_tiling_on_sc=False),
    scratch_types=(
        [
            pltpu.VMEM((K2OPS, 1, CPE), jnp.int32),
            pltpu.VMEM((K2OPS, 1, CPE), jnp.int32),
        ]
        + [pltpu.VMEM((CPE, DH), jnp.bfloat16) for _ in range(RING)]
        + [
            pltpu.VMEM((256, DH), jnp.bfloat16),
            pltpu.VMEM_SHARED((N, DH), jnp.bfloat16),
        ]
        + [pltpu.SemaphoreType.DMA for _ in range(RING + 1)]
    ),
)
def _scatter_kernel(src_hbm, dst_hbm, yl_hbm, yr_hbm, out_hbm,
                    src_v, dst_v, *rest):
    rows = rest[:RING]
    zbuf = rest[RING]
    s_sh = rest[RING + 1]
    gsems = rest[RING + 2:2 * RING + 2]
    ssem = rest[2 * RING + 2]
    c = lax.axis_index("c")
    s = lax.axis_index("s")
    pltpu.sync_copy(src_hbm.at[s], src_v)
    pltpu.sync_copy(dst_hbm.at[s], dst_v)
    def zb_body(i, carry):
        for k in range(DH // 32):
            zbuf[i, pl.ds(k * 32, 32)] = jnp.zeros((32,), jnp.bfloat16)
        return carry
    lax.fori_loop(0, 256, zb_body, 0)
    for q in range(4):
        pltpu.sync_copy(zbuf, s_sh.at[pl.ds(s * 1024 + q * 256, 256)])
    plsc.subcore_barrier()

    def run(table):
        def body(g, carry):
            @pl.when(g > 0)
            def _():
                for b in range(RING):
                    pltpu.make_async_copy(
                        table.at[pl.ds(0, CPE)], rows[b], ssem).wait()
            gd = [pltpu.async_copy(table.at[src_v.at[RING * g + b, 0]],
                                   rows[b], gsems[b]) for b in range(RING)]
            for b in range(RING):
                gd[b].wait()
                pltpu.async_copy(
                    rows[b], s_sh.at[dst_v.at[RING * g + b, 0]], ssem,
                    add=True)
            return carry
        lax.fori_loop(0, K2OPS // RING, body, 0)
        for b in range(RING):
            pltpu.make_async_copy(table.at[pl.ds(0, CPE)], rows[b],
                                  ssem).wait()

    @pl.when(c == 0)
    def _():
        run(yl_hbm)

    @pl.when(c == 1)
    def _():
        run(yr_hbm)

    plsc.subcore_barrier()
    for q in range(4):
        pltpu.sync_copy(s_sh.at[pl.ds(s * 1024 + q * 256, 256)], zbuf)
        pltpu.sync_copy(zbuf, out_hbm.at[c, pl.ds(s * 1024 + q * 256, 256)])


_BLK = 2048


def _t0_body(degp_ref, x_ref, yl_ref, yr_ref, dinv_ref):
    deg = (degp_ref[0, :, 0:1].astype(jnp.float32)
           + degp_ref[1, :, 0:1].astype(jnp.float32) + 1.0)
    dinv = lax.rsqrt(deg)
    y = (x_ref[...] * dinv).astype(jnp.bfloat16)
    yl_ref[...] = y[:, :DH]
    yr_ref[...] = y[:, DH:]
    dinv_ref[...] = dinv


def _t1_body(sl_ref, sr_ref, x_ref, dinv_ref, wl_ref, wr_ref,
             bg_ref, out_ref):
    dinv = dinv_ref[...]
    xs = x_ref[...] * dinv
    al = (sl_ref[...].astype(jnp.float32) + xs[:, :DH]) * dinv
    ar = (sr_ref[...].astype(jnp.float32) + xs[:, DH:]) * dinv
    dn = (((1,), (1,)), ((), ()))
    h = lax.dot_general(al.astype(jnp.bfloat16),
                        wl_ref[...].astype(jnp.bfloat16), dn,
                        preferred_element_type=jnp.float32)
    h = h + lax.dot_general(ar.astype(jnp.bfloat16),
                            wr_ref[...].astype(jnp.bfloat16), dn,
                            preferred_element_type=jnp.float32)
    out_ref[...] = jnp.maximum(h + bg_ref[...], 0.0)


_BJ = 512
_NJ = SIN // _BJ


def _t2_body(a_ref, w1_ref, b1_ref, w2_ref, b2_ref, out_ref, acc_ref):
    j = pl.program_id(0)
    dn = (((1,), (1,)), ((), ()))
    h = lax.dot_general(a_ref[...].astype(jnp.bfloat16),
                        w1_ref[...].astype(jnp.bfloat16), dn,
                        preferred_element_type=jnp.float32)
    h = jnp.maximum(h + b1_ref[...], 0.0)
    p = lax.dot_general(h.astype(jnp.bfloat16),
                        w2_ref[...].astype(jnp.bfloat16), dn,
                        preferred_element_type=jnp.float32)

    @pl.when(j == 0)
    def _():
        acc_ref[...] = p

    @pl.when(j > 0)
    def _():
        acc_ref[...] += p

    @pl.when(j == _NJ - 1)
    def _():
        z = acc_ref[...] + b2_ref[...]
        m = jnp.max(z, axis=-1, keepdims=True)
        e = jnp.exp(z - m)
        out_ref[...] = e / jnp.sum(e, axis=-1, keepdims=True)


def kernel(x, edge_index, batch, W_gcn, b_gcn, W1, b1, W2, b2):
    del batch
    src = edge_index[0].reshape(NS, K2OPS, 1, CPE)
    dst = edge_index[1].reshape(NS, K2OPS, 1, CPE)
    dst_k1 = edge_index[1].reshape(NC * NS, K1OPS, 1, CPE)

    degp = _deg_kernel(dst_k1)

    t0 = pl.pallas_call(
        _t0_body,
        grid=(N // _BLK,),
        in_specs=[
            pl.BlockSpec((NC, _BLK, 2), lambda i: (0, i, 0)),
            pl.BlockSpec((_BLK, D), lambda i: (i, 0)),
        ],
        out_specs=[
            pl.BlockSpec((_BLK, DH), lambda i: (i, 0)),
            pl.BlockSpec((_BLK, DH), lambda i: (i, 0)),
            pl.BlockSpec((_BLK, 1), lambda i: (i, 0)),
        ],
        out_shape=[
            jax.ShapeDtypeStruct((N, DH), jnp.bfloat16),
            jax.ShapeDtypeStruct((N, DH), jnp.bfloat16),
            jax.ShapeDtypeStruct((N, 1), jnp.float32),
        ],
    )
    yl, yr, dinv = t0(degp, x)

    s_par = _scatter_kernel(src, dst, yl, yr)

    t1 = pl.pallas_call(
        _t1_body,
        grid=(N // _BLK,),
        in_specs=[
            pl.BlockSpec((_BLK, DH), lambda i: (i, 0)),
            pl.BlockSpec((_BLK, DH), lambda i: (i, 0)),
            pl.BlockSpec((_BLK, D), lambda i: (i, 0)),
            pl.BlockSpec((_BLK, 1), lambda i: (i, 0)),
            pl.BlockSpec((D, DH), lambda i: (0, 0)),
            pl.BlockSpec((D, DH), lambda i: (0, 0)),
            pl.BlockSpec((1, D), lambda i: (0, 0)),
        ],
        out_specs=pl.BlockSpec((_BLK, D), lambda i: (i, 0)),
        out_shape=jax.ShapeDtypeStruct((N, D), jnp.float32),
    )
    hga = t1(s_par[0], s_par[1], x, dinv,
             W_gcn[:, :DH], W_gcn[:, DH:], b_gcn.reshape(1, D))

    a = hga.reshape(G, SIN)
    w2p = jnp.zeros((D, SIN), jnp.float32).at[:OUT].set(W2)
    b2p = jnp.full((1, D), -1e30, jnp.float32).at[0, :OUT].set(b2)

    t2 = pl.pallas_call(
        _t2_body,
        grid=(_NJ,),
        in_specs=[
            pl.BlockSpec((G, SIN), lambda j: (0, 0)),
            pl.BlockSpec((_BJ, SIN), lambda j: (j, 0)),
            pl.BlockSpec((1, _BJ), lambda j: (0, j)),
            pl.BlockSpec((D, _BJ), lambda j: (0, j)),
            pl.BlockSpec((1, D), lambda j: (0, 0)),
        ],
        out_specs=pl.BlockSpec((G, D), lambda j: (0, 0)),
        out_shape=jax.ShapeDtypeStruct((G, D), jnp.float32),
        scratch_shapes=[pltpu.VMEM((G, D), jnp.float32)],
    )
    out = t2(a, W1, b1.reshape(1, SIN), w2p, b2p)
    return out[:, :OUT]


# speedup vs baseline: 1.2277x; 1.2277x over previous
import functools

import jax
import jax.numpy as jnp
from jax import lax
from jax.experimental import pallas as pl
from jax.experimental.pallas import tpu as pltpu
from jax.experimental.pallas import tpu_sc as plsc

N = 16384
E = 262144
D = 128
DH = 64
G = 256
SIN = 8192
OUT = 10
NC = 2
NS = 16
CPE = 512
K2OPS = E // NS // CPE
K1OPS = E // (NC * NS) // CPE
RING = 2
DW = 16

_mesh = plsc.VectorSubcoreMesh(core_axis_name="c", subcore_axis_name="s")


def _fill(ref, rows, width, value):
    def body(i, carry):
        for k in range(width // 16):
            ref[i, pl.ds(k * 16, 16)] = jnp.full((16,), value, jnp.float32)
        return carry
    lax.fori_loop(0, rows, body, 0)


@functools.partial(
    pl.kernel,
    out_type=jax.ShapeDtypeStruct((NC, N, DW), jnp.bfloat16),
    mesh=_mesh,
    compiler_params=pltpu.CompilerParams(use_tc_tiling_on_sc=False),
    scratch_types=[
        pltpu.VMEM((K1OPS, 1, CPE), jnp.int32),
        pltpu.VMEM((CPE, DW), jnp.bfloat16),
        pltpu.VMEM((1024, DW), jnp.bfloat16),
        pltpu.VMEM_SHARED((N, DW), jnp.bfloat16),
        pltpu.SemaphoreType.DMA,
    ],
)
def _deg_kernel(dst_hbm, out_hbm, idx_v, ones_v, zbuf, deg_sh, sem0):
    c = lax.axis_index("c")
    s = lax.axis_index("s")
    tid = c * NS + s
    pltpu.sync_copy(dst_hbm.at[tid], idx_v)
    def zb_body(i, carry):
        zbuf[pl.ds(2 * i, 2), :] = jnp.zeros((2, DW), jnp.bfloat16)
        return carry
    lax.fori_loop(0, 512, zb_body, 0)
    pltpu.sync_copy(zbuf, deg_sh.at[pl.ds(s * 1024, 1024)])
    def ones_body(i, carry):
        ones_v[pl.ds(2 * i, 2), :] = jnp.ones((2, DW), jnp.bfloat16)
        return carry
    lax.fori_loop(0, CPE // 2, ones_body, 0)
    plsc.subcore_barrier()
    def body(g, carry):
        ds_ = [pltpu.async_copy(ones_v, deg_sh.at[idx_v.at[4 * g + b, 0]],
                                sem0, add=True) for b in range(4)]
        for d in ds_:
            d.wait()
        return carry
    lax.fori_loop(0, K1OPS // 4, body, 0)
    plsc.subcore_barrier()
    pltpu.sync_copy(deg_sh.at[pl.ds(s * 1024, 1024)], zbuf)
    pltpu.sync_copy(zbuf, out_hbm.at[c, pl.ds(s * 1024, 1024)])


@functools.partial(
    pl.kernel,
    out_type=jax.ShapeDtypeStruct((NC, N, DH), jnp.bfloat16),
    mesh=_mesh,
    compiler_params=pltpu.CompilerParams(use_tc_tiling_on_sc=False),
    scratch_types=(
        [
            pltpu.VMEM((K2OPS, 1, CPE), jnp.int32),
            pltpu.VMEM((K2OPS, 1, CPE), jnp.int32),
        ]
        + [pltpu.VMEM((CPE, DH), jnp.bfloat16) for _ in range(RING)]
        + [
            pltpu.VMEM((256, DH), jnp.bfloat16),
            pltpu.VMEM_SHARED((N, DH), jnp.bfloat16),
        ]
        + [pltpu.SemaphoreType.DMA for _ in range(RING + 1)]
    ),
)
def _scatter_kernel(src_hbm, dst_hbm, yl_hbm, yr_hbm, out_hbm,
                    src_v, dst_v, *rest):
    rows = rest[:RING]
    zbuf = rest[RING]
    s_sh = rest[RING + 1]
    gsems = rest[RING + 2:2 * RING + 2]
    ssem = rest[2 * RING + 2]
    c = lax.axis_index("c")
    s = lax.axis_index("s")
    pltpu.sync_copy(src_hbm.at[s], src_v)
    pltpu.sync_copy(dst_hbm.at[s], dst_v)
    def zb_body(i, carry):
        for k in range(DH // 32):
            zbuf[i, pl.ds(k * 32, 32)] = jnp.zeros((32,), jnp.bfloat16)
        return carry
    lax.fori_loop(0, 256, zb_body, 0)
    for q in range(4):
        pltpu.sync_copy(zbuf, s_sh.at[pl.ds(s * 1024 + q * 256, 256)])
    plsc.subcore_barrier()

    def run(table):
        def body(g, carry):
            @pl.when(g > 0)
            def _():
                for b in range(RING):
                    pltpu.make_async_copy(
                        table.at[pl.ds(0, CPE)], rows[b], ssem).wait()
            gd = [pltpu.async_copy(table.at[src_v.at[RING * g + b, 0]],
                                   rows[b], gsems[b]) for b in range(RING)]
            for b in range(RING):
                gd[b].wait()
                pltpu.async_copy(
                    rows[b], s_sh.at[dst_v.at[RING * g + b, 0]], ssem,
                    add=True)
            return carry
        lax.fori_loop(0, K2OPS // RING, body, 0)
        for b in range(RING):
            pltpu.make_async_copy(table.at[pl.ds(0, CPE)], rows[b],
                                  ssem).wait()

    @pl.when(c == 0)
    def _():
        run(yl_hbm)

    @pl.when(c == 1)
    def _():
        run(yr_hbm)

    plsc.subcore_barrier()
    for q in range(4):
        pltpu.sync_copy(s_sh.at[pl.ds(s * 1024 + q * 256, 256)], zbuf)
        pltpu.sync_copy(zbuf, out_hbm.at[c, pl.ds(s * 1024 + q * 256, 256)])


_BLK = 2048


def _t0_body(degp_ref, x_ref, yl_ref, yr_ref, dinv_ref):
    deg = (degp_ref[0, :, 0:1].astype(jnp.float32)
           + degp_ref[1, :, 0:1].astype(jnp.float32) + 1.0)
    dinv = lax.rsqrt(deg)
    y = (x_ref[...] * dinv).astype(jnp.bfloat16)
    yl_ref[...] = y[:, :DH]
    yr_ref[...] = y[:, DH:]
    dinv_ref[...] = dinv


def _t1_body(sl_ref, sr_ref, x_ref, dinv_ref, wl_ref, wr_ref,
             bg_ref, out_ref):
    dinv = dinv_ref[...]
    xs = x_ref[...] * dinv
    al = (sl_ref[...].astype(jnp.float32) + xs[:, :DH]) * dinv
    ar = (sr_ref[...].astype(jnp.float32) + xs[:, DH:]) * dinv
    dn = (((1,), (1,)), ((), ()))
    h = lax.dot_general(al.astype(jnp.bfloat16),
                        wl_ref[...].astype(jnp.bfloat16), dn,
                        preferred_element_type=jnp.float32)
    h = h + lax.dot_general(ar.astype(jnp.bfloat16),
                            wr_ref[...].astype(jnp.bfloat16), dn,
                            preferred_element_type=jnp.float32)
    out_ref[...] = jnp.maximum(h + bg_ref[...], 0.0)


_BJ = 512
_NJ = SIN // _BJ


def _t2_body(a_ref, w1_ref, b1_ref, w2_ref, b2_ref, out_ref, acc_ref):
    j = pl.program_id(0)
    dn = (((1,), (1,)), ((), ()))
    h = lax.dot_general(a_ref[...].astype(jnp.bfloat16),
                        w1_ref[...].astype(jnp.bfloat16), dn,
                        preferred_element_type=jnp.float32)
    h = jnp.maximum(h + b1_ref[...], 0.0)
    p = lax.dot_general(h.astype(jnp.bfloat16),
                        w2_ref[...].astype(jnp.bfloat16), dn,
                        preferred_element_type=jnp.float32)

    @pl.when(j == 0)
    def _():
        acc_ref[...] = p

    @pl.when(j > 0)
    def _():
        acc_ref[...] += p

    @pl.when(j == _NJ - 1)
    def _():
        z = acc_ref[...] + b2_ref[...]
        m = jnp.max(z, axis=-1, keepdims=True)
        e = jnp.exp(z - m)
        out_ref[...] = e / jnp.sum(e, axis=-1, keepdims=True)


def kernel(x, edge_index, batch, W_gcn, b_gcn, W1, b1, W2, b2):
    del batch
    src = edge_index[0].reshape(NS, K2OPS, 1, CPE)
    dst = edge_index[1].reshape(NS, K2OPS, 1, CPE)
    dst_k1 = edge_index[1].reshape(NC * NS, K1OPS, 1, CPE)

    degp = _deg_kernel(dst_k1)

    t0 = pl.pallas_call(
        _t0_body,
        grid=(N // _BLK,),
        in_specs=[
            pl.BlockSpec((NC, _BLK, DW), lambda i: (0, i, 0)),
            pl.BlockSpec((_BLK, D), lambda i: (i, 0)),
        ],
        out_specs=[
            pl.BlockSpec((_BLK, DH), lambda i: (i, 0)),
            pl.BlockSpec((_BLK, DH), lambda i: (i, 0)),
            pl.BlockSpec((_BLK, 1), lambda i: (i, 0)),
        ],
        out_shape=[
            jax.ShapeDtypeStruct((N, DH), jnp.bfloat16),
            jax.ShapeDtypeStruct((N, DH), jnp.bfloat16),
            jax.ShapeDtypeStruct((N, 1), jnp.float32),
        ],
    )
    yl, yr, dinv = t0(degp, x)

    s_par = _scatter_kernel(src, dst, yl, yr)

    t1 = pl.pallas_call(
        _t1_body,
        grid=(N // _BLK,),
        in_specs=[
            pl.BlockSpec((_BLK, DH), lambda i: (i, 0)),
            pl.BlockSpec((_BLK, DH), lambda i: (i, 0)),
            pl.BlockSpec((_BLK, D), lambda i: (i, 0)),
            pl.BlockSpec((_BLK, 1), lambda i: (i, 0)),
            pl.BlockSpec((D, DH), lambda i: (0, 0)),
            pl.BlockSpec((D, DH), lambda i: (0, 0)),
            pl.BlockSpec((1, D), lambda i: (0, 0)),
        ],
        out_specs=pl.BlockSpec((_BLK, D), lambda i: (i, 0)),
        out_shape=jax.ShapeDtypeStruct((N, D), jnp.float32),
    )
    hga = t1(s_par[0], s_par[1], x, dinv,
             W_gcn[:, :DH], W_gcn[:, DH:], b_gcn.reshape(1, D))

    a = hga.reshape(G, SIN)
    w2p = jnp.zeros((D, SIN), jnp.float32).at[:OUT].set(W2)
    b2p = jnp.full((1, D), -1e30, jnp.float32).at[0, :OUT].set(b2)

    t2 = pl.pallas_call(
        _t2_body,
        grid=(_NJ,),
        in_specs=[
            pl.BlockSpec((G, SIN), lambda j: (0, 0)),
            pl.BlockSpec((_BJ, SIN), lambda j: (j, 0)),
            pl.BlockSpec((1, _BJ), lambda j: (0, j)),
            pl.BlockSpec((D, _BJ), lambda j: (0, j)),
            pl.BlockSpec((1, D), lambda j: (0, 0)),
        ],
        out_specs=pl.BlockSpec((G, D), lambda j: (0, 0)),
        out_shape=jax.ShapeDtypeStruct((G, D), jnp.float32),
        scratch_shapes=[pltpu.VMEM((G, D), jnp.float32)],
    )
    out = t2(a, W1, b1.reshape(1, SIN), w2p, b2p)
    return out[:, :OUT]


# speedup vs baseline: 1.2285x; 1.0007x over previous
import functools

import jax
import jax.numpy as jnp
from jax import lax
from jax.experimental import pallas as pl
from jax.experimental.pallas import tpu as pltpu
from jax.experimental.pallas import tpu_sc as plsc

N = 16384
E = 262144
D = 128
DH = 64
G = 256
SIN = 8192
OUT = 10
NC = 2
NS = 16
CPE = 512
K2OPS = E // NS // CPE
K1OPS = E // (NC * NS) // CPE
RING = 2
DW = 16

_mesh = plsc.VectorSubcoreMesh(core_axis_name="c", subcore_axis_name="s",
                               num_cores=NC, num_subcores=NS)


@functools.partial(
    pl.kernel,
    out_type=jax.ShapeDtypeStruct((NC, N, DW), jnp.bfloat16),
    mesh=_mesh,
    compiler_params=pltpu.CompilerParams(use_tc_tiling_on_sc=False),
    scratch_types=[
        pltpu.VMEM((K1OPS, 1, CPE), jnp.int32),
        pltpu.VMEM((CPE, DW), jnp.bfloat16),
        pltpu.VMEM((1024, DW), jnp.bfloat16),
        pltpu.VMEM_SHARED((N, DW), jnp.bfloat16),
        pltpu.SemaphoreType.DMA,
    ],
)
def _deg_kernel(dst_hbm, out_hbm, idx_v, ones_v, zbuf, deg_sh, sem0):
    c = lax.axis_index("c")
    s = lax.axis_index("s")
    tid = c * NS + s
    pltpu.sync_copy(dst_hbm.at[tid], idx_v)
    def zb_body(i, carry):
        zbuf[pl.ds(2 * i, 2), :] = jnp.zeros((2, DW), jnp.bfloat16)
        return carry
    lax.fori_loop(0, 512, zb_body, 0)
    pltpu.sync_copy(zbuf, deg_sh.at[pl.ds(s * 1024, 1024)])
    def ones_body(i, carry):
        ones_v[pl.ds(2 * i, 2), :] = jnp.ones((2, DW), jnp.bfloat16)
        return carry
    lax.fori_loop(0, CPE // 2, ones_body, 0)
    plsc.subcore_barrier()
    def body(g, carry):
        ds_ = [pltpu.async_copy(ones_v, deg_sh.at[idx_v.at[4 * g + b, 0]],
                                sem0, add=True) for b in range(4)]
        for d in ds_:
            d.wait()
        return carry
    lax.fori_loop(0, K1OPS // 4, body, 0)
    plsc.subcore_barrier()
    pltpu.sync_copy(deg_sh.at[pl.ds(s * 1024, 1024)], zbuf)
    pltpu.sync_copy(zbuf, out_hbm.at[c, pl.ds(s * 1024, 1024)])


@functools.partial(
    pl.kernel,
    out_type=jax.ShapeDtypeStruct((NC, N, DH), jnp.bfloat16),
    mesh=_mesh,
    compiler_params=pltpu.CompilerParams(use_tc_tiling_on_sc=False),
    scratch_types=(
        [
            pltpu.VMEM((K2OPS, 1, CPE), jnp.int32),
            pltpu.VMEM((K2OPS, 1, CPE), jnp.int32),
        ]
        + [pltpu.VMEM((CPE, DH), jnp.bfloat16) for _ in range(RING)]
        + [
            pltpu.VMEM((256, DH), jnp.bfloat16),
            pltpu.VMEM_SHARED((N, DH), jnp.bfloat16),
        ]
        + [pltpu.SemaphoreType.DMA for _ in range(RING + 1)]
    ),
)
def _scatter_kernel(src_hbm, dst_hbm, yl_hbm, yr_hbm, out_hbm,
                    src_v, dst_v, *rest):
    rows = rest[:RING]
    zbuf = rest[RING]
    s_sh = rest[RING + 1]
    gsems = rest[RING + 2:2 * RING + 2]
    ssem = rest[2 * RING + 2]
    c = lax.axis_index("c")
    s = lax.axis_index("s")
    pltpu.sync_copy(src_hbm.at[s], src_v)
    pltpu.sync_copy(dst_hbm.at[s], dst_v)
    def zb_body(i, carry):
        for k in range(DH // 32):
            zbuf[i, pl.ds(k * 32, 32)] = jnp.zeros((32,), jnp.bfloat16)
        return carry
    lax.fori_loop(0, 256, zb_body, 0)
    for q in range(4):
        pltpu.sync_copy(zbuf, s_sh.at[pl.ds(s * 1024 + q * 256, 256)])
    plsc.subcore_barrier()

    def run(table):
        def body(g, carry):
            @pl.when(g > 0)
            def _():
                for b in range(RING):
                    pltpu.make_async_copy(
                        table.at[pl.ds(0, CPE)], rows[b], ssem).wait()
            gd = [pltpu.async_copy(table.at[src_v.at[RING * g + b, 0]],
                                   rows[b], gsems[b]) for b in range(RING)]
            for b in range(RING):
                gd[b].wait()
                pltpu.async_copy(
                    rows[b], s_sh.at[dst_v.at[RING * g + b, 0]], ssem,
                    add=True)
            return carry
        lax.fori_loop(0, K2OPS // RING, body, 0)
        for b in range(RING):
            pltpu.make_async_copy(table.at[pl.ds(0, CPE)], rows[b],
                                  ssem).wait()

    @pl.when(c == 0)
    def _():
        run(yl_hbm)

    @pl.when(c == 1)
    def _():
        run(yr_hbm)

    plsc.subcore_barrier()
    for q in range(4):
        pltpu.sync_copy(s_sh.at[pl.ds(s * 1024 + q * 256, 256)], zbuf)
        pltpu.sync_copy(zbuf, out_hbm.at[c, pl.ds(s * 1024 + q * 256, 256)])


_BLK = 2048


def _t0_body(degp_ref, x_ref, yl_ref, yr_ref, dinv_ref):
    deg = (degp_ref[0, :, 0:1].astype(jnp.float32)
           + degp_ref[1, :, 0:1].astype(jnp.float32) + 1.0)
    dinv = lax.rsqrt(deg)
    y = (x_ref[...] * dinv).astype(jnp.bfloat16)
    yl_ref[...] = y[:, :DH]
    yr_ref[...] = y[:, DH:]
    dinv_ref[...] = dinv


def _t1_body(sl_ref, sr_ref, x_ref, dinv_ref, wl_ref, wr_ref,
             bg_ref, out_ref):
    dinv = dinv_ref[...]
    xs = x_ref[...] * dinv
    al = (sl_ref[...].astype(jnp.float32) + xs[:, :DH]) * dinv
    ar = (sr_ref[...].astype(jnp.float32) + xs[:, DH:]) * dinv
    dn = (((1,), (1,)), ((), ()))
    h = lax.dot_general(al.astype(jnp.bfloat16),
                        wl_ref[...].astype(jnp.bfloat16), dn,
                        preferred_element_type=jnp.float32)
    h = h + lax.dot_general(ar.astype(jnp.bfloat16),
                            wr_ref[...].astype(jnp.bfloat16), dn,
                            preferred_element_type=jnp.float32)
    out_ref[...] = jnp.maximum(h + bg_ref[...], 0.0)


_BJ = 512
_NJ = SIN // _BJ


def _t2_body(a_ref, w1_ref, b1_ref, w2_ref, b2_ref, out_ref, acc_ref):
    j = pl.program_id(0)
    dn = (((1,), (1,)), ((), ()))
    h = lax.dot_general(a_ref[...].astype(jnp.bfloat16),
                        w1_ref[...].astype(jnp.bfloat16), dn,
                        preferred_element_type=jnp.float32)
    h = jnp.maximum(h + b1_ref[...], 0.0)
    p = lax.dot_general(h.astype(jnp.bfloat16),
                        w2_ref[...].astype(jnp.bfloat16), dn,
                        preferred_element_type=jnp.float32)

    @pl.when(j == 0)
    def _():
        acc_ref[...] = p

    @pl.when(j > 0)
    def _():
        acc_ref[...] += p

    @pl.when(j == _NJ - 1)
    def _():
        z = acc_ref[...] + b2_ref[...]
        m = jnp.max(z, axis=-1, keepdims=True)
        e = jnp.exp(z - m)
        out_ref[...] = e / jnp.sum(e, axis=-1, keepdims=True)


def kernel(x, edge_index, batch, W_gcn, b_gcn, W1, b1, W2, b2):
    del batch
    src = edge_index[0].reshape(NS, K2OPS, 1, CPE)
    dst = edge_index[1].reshape(NS, K2OPS, 1, CPE)
    dst_k1 = edge_index[1].reshape(NC * NS, K1OPS, 1, CPE)

    degp = _deg_kernel(dst_k1)

    t0 = pl.pallas_call(
        _t0_body,
        grid=(N // _BLK,),
        in_specs=[
            pl.BlockSpec((NC, _BLK, DW), lambda i: (0, i, 0)),
            pl.BlockSpec((_BLK, D), lambda i: (i, 0)),
        ],
        out_specs=[
            pl.BlockSpec((_BLK, DH), lambda i: (i, 0)),
            pl.BlockSpec((_BLK, DH), lambda i: (i, 0)),
            pl.BlockSpec((_BLK, 1), lambda i: (i, 0)),
        ],
        out_shape=[
            jax.ShapeDtypeStruct((N, DH), jnp.bfloat16),
            jax.ShapeDtypeStruct((N, DH), jnp.bfloat16),
            jax.ShapeDtypeStruct((N, 1), jnp.float32),
        ],
    )
    yl, yr, dinv = t0(degp, x)

    s_par = _scatter_kernel(src, dst, yl, yr)

    t1 = pl.pallas_call(
        _t1_body,
        grid=(N // _BLK,),
        in_specs=[
            pl.BlockSpec((_BLK, DH), lambda i: (i, 0)),
            pl.BlockSpec((_BLK, DH), lambda i: (i, 0)),
            pl.BlockSpec((_BLK, D), lambda i: (i, 0)),
            pl.BlockSpec((_BLK, 1), lambda i: (i, 0)),
            pl.BlockSpec((D, DH), lambda i: (0, 0)),
            pl.BlockSpec((D, DH), lambda i: (0, 0)),
            pl.BlockSpec((1, D), lambda i: (0, 0)),
        ],
        out_specs=pl.BlockSpec((_BLK, D), lambda i: (i, 0)),
        out_shape=jax.ShapeDtypeStruct((N, D), jnp.float32),
    )
    hga = t1(s_par[0], s_par[1], x, dinv,
             W_gcn[:, :DH], W_gcn[:, DH:], b_gcn.reshape(1, D))

    a = hga.reshape(G, SIN)
    w2p = jnp.zeros((D, SIN), jnp.float32).at[:OUT].set(W2)
    b2p = jnp.full((1, D), -1e30, jnp.float32).at[0, :OUT].set(b2)

    t2 = pl.pallas_call(
        _t2_body,
        grid=(_NJ,),
        in_specs=[
            pl.BlockSpec((G, SIN), lambda j: (0, 0)),
            pl.BlockSpec((_BJ, SIN), lambda j: (j, 0)),
            pl.BlockSpec((1, _BJ), lambda j: (0, j)),
            pl.BlockSpec((D, _BJ), lambda j: (0, j)),
            pl.BlockSpec((1, D), lambda j: (0, 0)),
        ],
        out_specs=pl.BlockSpec((G, D), lambda j: (0, 0)),
        out_shape=jax.ShapeDtypeStruct((G, D), jnp.float32),
        scratch_shapes=[pltpu.VMEM((G, D), jnp.float32)],
    )
    out = t2(a, W1, b1.reshape(1, SIN), w2p, b2p)
    return out[:, :OUT]
